# baseline (speedup 1.0000x reference)
import functools

import jax
import jax.numpy as jnp
from jax import lax
from jax.experimental import pallas as pl
from jax.experimental.pallas import tpu as pltpu
from jax.experimental.pallas import tpu_sc as plsc

_N = 10000
_D = 128
_E = 320000
_E2 = 640000
_NPAD = 10240
_TRASH = _N
_CH = 80
_NC = 2
_NS = 16
_NW = _NC * _NS
_DSEG = _NPAD // _NS
_BPAD = 30720

_f32 = jnp.float32
_i32 = jnp.int32

_MESH = plsc.VectorSubcoreMesh(core_axis_name="c", subcore_axis_name="s")


def _zero_vec(ref, nelem):
    def zb(i, carry):
        ref[pl.ds(i * 16, 16)] = jnp.zeros((16,), _f32)
        return carry
    lax.fori_loop(0, nelem // 16, zb, None)


def _body_a(cd, degp, cs, ones_v, zdeg, deg_sh, ds0, ds1):
    c = lax.axis_index("c")
    s = lax.axis_index("s")
    wid = c * _NS + s
    rpt = _E // _NW // _CH
    pltpu.sync_copy(cd.at[wid], cs)
    for j in range(_CH // 16):
        ones_v[pl.ds(j * 16, 16)] = jnp.ones((16,), _f32)
    _zero_vec(zdeg, _DSEG)
    pltpu.sync_copy(zdeg, deg_sh.at[pl.ds(s * _DSEG, _DSEG)])
    plsc.subcore_barrier()

    pltpu.async_copy(ones_v, deg_sh.at[cs.at[0]], ds0, add=True)
    pltpu.async_copy(ones_v, deg_sh.at[cs.at[1]], ds1, add=True)

    def pair(k, carry):
        i2 = 2 * k + 2
        i3 = 2 * k + 3

        @pl.when(i2 < rpt)
        def _():
            pltpu.make_async_copy(ones_v, deg_sh.at[cs.at[0]], ds0).wait()
            pltpu.async_copy(ones_v, deg_sh.at[cs.at[i2]], ds0, add=True)

        @pl.when(i3 < rpt)
        def _():
            pltpu.make_async_copy(ones_v, deg_sh.at[cs.at[0]], ds1).wait()
            pltpu.async_copy(ones_v, deg_sh.at[cs.at[i3]], ds1, add=True)

        return carry

    lax.fori_loop(0, rpt // 2, pair, None)
    pltpu.make_async_copy(ones_v, deg_sh.at[cs.at[0]], ds0).wait()
    pltpu.make_async_copy(ones_v, deg_sh.at[cs.at[0]], ds1).wait()
    plsc.subcore_barrier()
    pltpu.sync_copy(deg_sh.at[pl.ds(s * _DSEG, _DSEG)],
                    degp.at[pl.ds(c * _NPAD + s * _DSEG, _DSEG)])


@functools.partial(
    pl.kernel,
    out_type=jax.ShapeDtypeStruct((_NC * _NPAD,), _f32),
    mesh=_MESH,
    scratch_types=[
        pltpu.VMEM((_E // _NW // _CH, _CH), _i32),
        pltpu.VMEM((_CH,), _f32),
        pltpu.VMEM((_DSEG,), _f32),
        pltpu.VMEM_SHARED((_NPAD,), _f32),
        pltpu.SemaphoreType.DMA,
        pltpu.SemaphoreType.DMA,
    ],
)
def _kernel_a(cd, degp, *rest):
    _body_a(cd, degp, *rest)


def _body_b(rd, cd, mwin, col2, deg2p, rs, cs, c2s, c2r, key0, win0, key1,
            win1, keep0, keep1, zdeg, deg_sh, gs0, gs1, ds0, ds1, ds2, ds3):
    c = lax.axis_index("c")
    s = lax.axis_index("s")
    wid = c * _NS + s
    rpt = _E // _NW // _CH
    pltpu.sync_copy(rd.at[wid], rs)
    pltpu.sync_copy(cd.at[wid], cs)
    _zero_vec(zdeg, _DSEG)
    pltpu.sync_copy(zdeg, deg_sh.at[pl.ds(s * _DSEG, _DSEG)])
    plsc.subcore_barrier()
    ebase = wid * (_E // _NW)

    def do_gather(i, key_v, win_v, gsem):
        for j in range(_CH // 16):
            r16 = rs[i, pl.ds(j * 16, 16)]
            c16 = cs[i, pl.ds(j * 16, 16)]
            key_v[pl.ds(j * 16, 16)] = (jnp.minimum(r16, c16) * _N
                                        + jnp.maximum(r16, c16))
        pltpu.async_copy(mwin.at[key_v], win_v, gsem)

    def process(i, win_v, keep_v, dsc, dsr):
        for j in range(_CH // 16):
            r16 = rs[i, pl.ds(j * 16, 16)]
            c16 = cs[i, pl.ds(j * 16, 16)]
            w16 = win_v[pl.ds(j * 16, 16)]
            e16 = lax.iota(_i32, 16) + (ebase + i * _CH + j * 16)
            keep16 = (w16 == e16) & (r16 != c16)
            c2s[pl.ds(i * _CH + j * 16, 16)] = jnp.where(keep16, c16, _TRASH)
            c2r[pl.ds(i * _CH + j * 16, 16)] = jnp.where(keep16, r16, _TRASH)
            keep_v[pl.ds(j * 16, 16)] = jnp.where(keep16, 1.0, 0.0).astype(_f32)
        pltpu.async_copy(keep_v, deg_sh.at[cs.at[i]], dsc, add=True)
        pltpu.async_copy(keep_v, deg_sh.at[rs.at[i]], dsr, add=True)

    do_gather(0, key0, win0, gs0)
    do_gather(1, key1, win1, gs1)

    def pair(k, carry):
        i0 = 2 * k
        i1 = 2 * k + 1
        i2 = 2 * k + 2
        i3 = 2 * k + 3

        @pl.when(k > 0)
        def _():
            pltpu.make_async_copy(keep0, deg_sh.at[cs.at[0]], ds0).wait()
            pltpu.make_async_copy(keep0, deg_sh.at[cs.at[0]], ds2).wait()
            pltpu.make_async_copy(keep1, deg_sh.at[cs.at[0]], ds1).wait()
            pltpu.make_async_copy(keep1, deg_sh.at[cs.at[0]], ds3).wait()

        pltpu.make_async_copy(mwin.at[key0], win0, gs0).wait()
        process(i0, win0, keep0, ds0, ds2)

        @pl.when(i2 < rpt)
        def _():
            do_gather(i2, key0, win0, gs0)

        pltpu.make_async_copy(mwin.at[key1], win1, gs1).wait()
        process(i1, win1, keep1, ds1, ds3)

        @pl.when(i3 < rpt)
        def _():
            do_gather(i3, key1, win1, gs1)

        return carry

    lax.fori_loop(0, rpt // 2, pair, None)
    pltpu.make_async_copy(keep0, deg_sh.at[cs.at[0]], ds0).wait()
    pltpu.make_async_copy(keep0, deg_sh.at[cs.at[0]], ds2).wait()
    pltpu.make_async_copy(mwin.at[key0], win0, gs0).wait()
    process(rpt - 1, win0, keep0, ds0, ds2)
    pltpu.make_async_copy(keep0, deg_sh.at[cs.at[0]], ds0).wait()
    pltpu.make_async_copy(keep0, deg_sh.at[cs.at[0]], ds2).wait()
    pltpu.make_async_copy(keep1, deg_sh.at[cs.at[0]], ds1).wait()
    pltpu.make_async_copy(keep1, deg_sh.at[cs.at[0]], ds3).wait()
    ept = _E // _NW
    pltpu.sync_copy(c2s, col2.at[pl.ds(wid * ept, ept)])
    pltpu.sync_copy(c2r, col2.at[pl.ds(_E + wid * ept, ept)])
    plsc.subcore_barrier()
    pltpu.sync_copy(deg_sh.at[pl.ds(s * _DSEG, _DSEG)],
                    deg2p.at[pl.ds(c * _NPAD + s * _DSEG, _DSEG)])


@functools.partial(
    pl.kernel,
    out_type=(
        jax.ShapeDtypeStruct((_E2,), _i32),
        jax.ShapeDtypeStruct((_NC * _NPAD,), _f32),
    ),
    mesh=_MESH,
    scratch_types=[
        pltpu.VMEM((_E // _NW // _CH, _CH), _i32),
        pltpu.VMEM((_E // _NW // _CH, _CH), _i32),
        pltpu.VMEM((_E // _NW,), _i32),
        pltpu.VMEM((_E // _NW,), _i32),
        pltpu.VMEM((_CH,), _i32),
        pltpu.VMEM((_CH,), _i32),
        pltpu.VMEM((_CH,), _i32),
        pltpu.VMEM((_CH,), _i32),
        pltpu.VMEM((_CH,), _f32),
        pltpu.VMEM((_CH,), _f32),
        pltpu.VMEM((_DSEG,), _f32),
        pltpu.VMEM_SHARED((_NPAD,), _f32),
        pltpu.SemaphoreType.DMA,
        pltpu.SemaphoreType.DMA,
        pltpu.SemaphoreType.DMA,
        pltpu.SemaphoreType.DMA,
        pltpu.SemaphoreType.DMA,
        pltpu.SemaphoreType.DMA,
    ],
)
def _kernel_b(rd, cd, mwin, col2, deg2p, *rest):
    _body_b(rd, cd, mwin, col2, deg2p, *rest)


def _make_scat(n_edges, split_rows, emit_win=False):
    ept = n_edges // _NW
    rpt = ept // _CH

    def body(rlo, rhi, colf, g, accp, *rest):
        if emit_win:
            (mwin, rv0, cv0, rv1, cv1, cvs0, cvs1, buf0, buf1, zrows, acc_sh,
             gs0, gs1, ss0, ss1, irs0, irs1, ics0, ics1,
             key0, ids0, key1, ids1, ws0, ws1) = rest
        else:
            (rv0, cv0, rv1, cv1, cvs0, cvs1, buf0, buf1, zrows, acc_sh,
             gs0, gs1, ss0, ss1, irs0, irs1, ics0, ics1) = rest
        c = lax.axis_index("c")
        s = lax.axis_index("s")
        wid = c * _NS + s

        def zr(i, carry):
            for j in range(_D // 16):
                zrows[i, pl.ds(j * 16, 16)] = jnp.zeros((16,), _f32)
            return carry

        lax.fori_loop(0, 16, zr, None)
        for k in range(_DSEG // 16):
            pltpu.sync_copy(zrows, acc_sh.at[pl.ds(s * _DSEG + k * 16, 16)])
        plsc.subcore_barrier()
        ebase = wid * ept

        def idx_load(i, rv, cv, irs, ics):
            if split_rows:
                hbase = s * ept + i * _CH

                @pl.when(c == 0)
                def _():
                    pltpu.async_copy(rlo.at[pl.ds(hbase, _CH)], rv, irs)

                @pl.when(c == 1)
                def _():
                    pltpu.async_copy(rhi.at[pl.ds(hbase, _CH)], rv, irs)
            else:
                pltpu.async_copy(rlo.at[pl.ds(ebase + i * _CH, _CH)], rv, irs)
            pltpu.async_copy(colf.at[pl.ds(ebase + i * _CH, _CH)], cv, ics)

        def idx_wait(rv, cv, irs, ics):
            pltpu.make_async_copy(rlo.at[pl.ds(ebase, _CH)], rv, irs).wait()
            pltpu.make_async_copy(colf.at[pl.ds(ebase, _CH)], cv, ics).wait()

        def load_and_gather(i, rv, cv, buf, irs, ics, gsem):
            idx_load(i, rv, cv, irs, ics)
            idx_wait(rv, cv, irs, ics)
            pltpu.async_copy(g.at[rv], buf, gsem)

        def cv_save(cv, cvs):
            for j in range(_CH // 16):
                cvs[pl.ds(j * 16, 16)] = cv[pl.ds(j * 16, 16)]

        def win_scatter(i, rv, cv, key_v, ids_v, wsem):
            for j in range(_CH // 16):
                r16 = rv[pl.ds(j * 16, 16)]
                c16 = cv[pl.ds(j * 16, 16)]
                key_v[pl.ds(j * 16, 16)] = (jnp.minimum(r16, c16) * _N
                                            + jnp.maximum(r16, c16))
                ids_v[pl.ds(j * 16, 16)] = (lax.iota(_i32, 16)
                                            + (ebase + i * _CH + j * 16))
            pltpu.async_copy(ids_v, mwin.at[key_v], wsem)

        load_and_gather(0, rv0, cv0, buf0, irs0, ics0, gs0)
        load_and_gather(1, rv1, cv1, buf1, irs1, ics1, gs1)

        def pair(k, carry):
            i0 = 2 * k
            i1 = 2 * k + 1
            i2 = 2 * k + 2
            i3 = 2 * k + 3
            if emit_win:
                @pl.when(k > 0)
                def _():
                    pltpu.make_async_copy(ids0, mwin.at[key0], ws0).wait()
                    pltpu.make_async_copy(ids1, mwin.at[key1], ws1).wait()

            pltpu.make_async_copy(g.at[rv0], buf0, gs0).wait()
            cv_save(cv0, cvs0)
            pltpu.async_copy(buf0, acc_sh.at[cvs0], ss0, add=True)
            if emit_win:
                win_scatter(i0, rv0, cv0, key0, ids0, ws0)

            @pl.when(i2 < rpt)
            def _():
                idx_load(i2, rv0, cv0, irs0, ics0)

            pltpu.make_async_copy(g.at[rv1], buf1, gs1).wait()
            cv_save(cv1, cvs1)
            pltpu.async_copy(buf1, acc_sh.at[cvs1], ss1, add=True)
            if emit_win:
                win_scatter(i1, rv1, cv1, key1, ids1, ws1)

            @pl.when(i3 < rpt)
            def _():
                idx_load(i3, rv1, cv1, irs1, ics1)

            @pl.when(i2 < rpt)
            def _():
                idx_wait(rv0, cv0, irs0, ics0)
                pltpu.make_async_copy(buf0, acc_sh.at[cvs0], ss0).wait()
                pltpu.async_copy(g.at[rv0], buf0, gs0)

            @pl.when(i3 < rpt)
            def _():
                idx_wait(rv1, cv1, irs1, ics1)
                pltpu.make_async_copy(buf1, acc_sh.at[cvs1], ss1).wait()
                pltpu.async_copy(g.at[rv1], buf1, gs1)

            return carry

        lax.fori_loop(0, rpt // 2, pair, None)
        if rpt % 2 == 1:
            pltpu.make_async_copy(g.at[rv0], buf0, gs0).wait()
            pltpu.sync_copy(buf0, acc_sh.at[cv0], add=True)
            pltpu.make_async_copy(buf1, acc_sh.at[cvs1], ss1).wait()
            if emit_win:
                pltpu.make_async_copy(ids0, mwin.at[key0], ws0).wait()
                win_scatter(rpt - 1, rv0, cv0, key0, ids0, ws0)
        else:
            pltpu.make_async_copy(buf0, acc_sh.at[cvs0], ss0).wait()
            pltpu.make_async_copy(buf1, acc_sh.at[cvs1], ss1).wait()
        if emit_win:
            pltpu.make_async_copy(ids0, mwin.at[key0], ws0).wait()
            pltpu.make_async_copy(ids1, mwin.at[key1], ws1).wait()

        plsc.subcore_barrier()
        pltpu.sync_copy(acc_sh.at[pl.ds(s * _DSEG, _DSEG)],
                        accp.at[c, pl.ds(s * _DSEG, _DSEG)])

    out_type = jax.ShapeDtypeStruct((_NC, _NPAD, _D), _f32)
    scratch = [
        pltpu.VMEM((_CH,), _i32),
        pltpu.VMEM((_CH,), _i32),
        pltpu.VMEM((_CH,), _i32),
        pltpu.VMEM((_CH,), _i32),
        pltpu.VMEM((_CH,), _i32),
        pltpu.VMEM((_CH,), _i32),
        pltpu.VMEM((_CH, _D), _f32),
        pltpu.VMEM((_CH, _D), _f32),
        pltpu.VMEM((16, _D), _f32),
        pltpu.VMEM_SHARED((_NPAD, _D), _f32),
        pltpu.SemaphoreType.DMA,
        pltpu.SemaphoreType.DMA,
        pltpu.SemaphoreType.DMA,
        pltpu.SemaphoreType.DMA,
        pltpu.SemaphoreType.DMA,
        pltpu.SemaphoreType.DMA,
        pltpu.SemaphoreType.DMA,
        pltpu.SemaphoreType.DMA,
    ]
    if emit_win:
        out_type = (out_type, jax.ShapeDtypeStruct((_N * _N,), _i32))
        scratch = scratch + [
            pltpu.VMEM((_CH,), _i32),
            pltpu.VMEM((_CH,), _i32),
            pltpu.VMEM((_CH,), _i32),
            pltpu.VMEM((_CH,), _i32),
            pltpu.SemaphoreType.DMA,
            pltpu.SemaphoreType.DMA,
        ]
    return pl.kernel(
        body,
        out_type=out_type,
        mesh=_MESH,
        scratch_types=scratch,
    )


_scat_e_win = _make_scat(_E, False, emit_win=True)
_scat_e = _make_scat(_E, False)
_scat_e2 = _make_scat(_E2, True)


def _body_g(src, idxd, out, is_, buf, sem):
    c = lax.axis_index("c")
    s = lax.axis_index("s")
    wid = c * _NS + s
    rpt = _BPAD // _NW // _CH
    slab0 = wid * rpt
    pltpu.sync_copy(idxd.at[wid], is_)

    def body(i, carry):
        pltpu.async_copy(src.at[is_.at[i]], buf, sem).wait()
        pltpu.sync_copy(buf, out.at[pl.ds((slab0 + i) * _CH, _CH)])
        return carry

    lax.fori_loop(0, rpt, body, None)


@functools.partial(
    pl.kernel,
    out_type=jax.ShapeDtypeStruct((_BPAD, _D), _f32),
    mesh=_MESH,
    scratch_types=[
        pltpu.VMEM((_BPAD // _NW // _CH, _CH), _i32),
        pltpu.VMEM((_CH, _D), _f32),
        pltpu.SemaphoreType.DMA,
    ],
)
def _kernel_g(src, idxd, out, *rest):
    _body_g(src, idxd, out, *rest)


_BLK = 2000


def _mm1_body(x_ref, w_ref, deg_ref, o_ref):
    dinv = lax.rsqrt(deg_ref[...])
    o_ref[...] = dinv * jnp.dot(
        x_ref[...], w_ref[...], preferred_element_type=_f32)


def _comb1_body(acc_ref, g1_ref, deg_ref, b_ref, w_ref, o_ref):
    dinv = lax.rsqrt(deg_ref[...])
    sacc = acc_ref[0] + acc_ref[1] + g1_ref[...]
    u = jnp.maximum(dinv * sacc + b_ref[...], 0.0)
    o_ref[...] = dinv * jnp.dot(
        u, w_ref[...], preferred_element_type=_f32)


def _comb2_body(acc_ref, g2_ref, deg_ref, b2_ref, wp_ref, bp_ref, deg2_ref, o_ref):
    dinv = lax.rsqrt(deg_ref[...])
    deg2 = deg2_ref[...]
    dinv2 = jnp.where(deg2 > 0, lax.rsqrt(deg2), 0.0)
    sacc = acc_ref[0] + acc_ref[1] + g2_ref[...]
    v = dinv * sacc + b2_ref[...]
    nodeb = jnp.dot(v, wp_ref[...], preferred_element_type=_f32) + bp_ref[...]
    o_ref[...] = dinv2 * nodeb


def _h1_body(acc_ref, deg2_ref, o_ref):
    deg2 = deg2_ref[...]
    ideg2 = jnp.where(deg2 > 0, 1.0 / deg2, 0.0)
    o_ref[...] = ideg2 * (acc_ref[0] + acc_ref[1])


def _h2_body(acc_ref, o_ref):
    o_ref[...] = acc_ref[0] + acc_ref[1]


def _row_spec():
    return pl.BlockSpec((_BLK, _D), lambda i: (i, 0))


def _acc_spec():
    return pl.BlockSpec((_NC, _BLK, _D), lambda i: (0, i, 0))


def _w_spec():
    return pl.BlockSpec((_D, _D), lambda i: (0, 0))


def _b_spec():
    return pl.BlockSpec((1, _D), lambda i: (0, 0))


def _deg_spec():
    return pl.BlockSpec((_BLK, 1), lambda i: (i, 0))


def _loss_body(a_ref, p_ref, g_ref, out_ref):
    i = pl.program_id(0)
    a = a_ref[...]
    p = p_ref[...]
    g = g_ref[...]
    na = jnp.maximum(jnp.sqrt(jnp.sum(a * a, axis=-1, keepdims=True)), 1e-8)
    npp = jnp.maximum(jnp.sqrt(jnp.sum(p * p, axis=-1, keepdims=True)), 1e-8)
    ng = jnp.maximum(jnp.sqrt(jnp.sum(g * g, axis=-1, keepdims=True)), 1e-8)
    cx = jnp.sum(a * p, axis=-1, keepdims=True) / (na * npp)
    cy = jnp.sum(a * g, axis=-1, keepdims=True) / (na * ng)
    li = jnp.log(1.0 + jnp.exp((cy - cx) / 0.2))

    @pl.when(i == 0)
    def _():
        out_ref[0, 0] = 0.0

    out_ref[0, 0] += jnp.sum(li)


def _loss(gath, b):
    nb = _N // _BLK
    out = pl.pallas_call(
        _loss_body,
        grid=(nb,),
        in_specs=[
            pl.BlockSpec((_BLK, _D), lambda i: (i, 0)),
            pl.BlockSpec((_BLK, _D), lambda i: (i + nb, 0)),
            pl.BlockSpec((_BLK, _D), lambda i: (i + 2 * nb, 0)),
        ],
        out_specs=pl.BlockSpec(memory_space=pltpu.SMEM),
        out_shape=jax.ShapeDtypeStruct((1, 1), _f32),
    )(gath, gath, gath)
    return out[0, 0] / b


def kernel(x, edge_index, batch, W1, b1, W2, b2, Wp, bp):
    n = _N
    ei = edge_index.astype(_i32)
    rd = ei[0].reshape(_NW, _E // _NW // _CH, _CH)
    cd = ei[1].reshape(_NW, _E // _NW // _CH, _CH)
    rowf = ei[0]
    colf = ei[1]

    degp = _kernel_a(cd)
    degp = degp.reshape(_NC, _NPAD)
    deg1 = (degp[0, :n] + degp[1, :n] + 1.0).reshape(n, 1)

    g1 = pl.pallas_call(
        _mm1_body, grid=(n // _BLK,),
        in_specs=[_row_spec(), _w_spec(), _deg_spec()],
        out_specs=_row_spec(),
        out_shape=jax.ShapeDtypeStruct((n, _D), _f32),
    )(x, W1, deg1)

    acc1, mwin = _scat_e_win(rowf, rowf, colf, g1)

    col2f, deg2p = _kernel_b(rd, cd, mwin)
    deg2p = deg2p.reshape(_NC, _NPAD)
    deg2 = (deg2p[0, :n] + deg2p[1, :n]).reshape(n, 1)

    g2 = pl.pallas_call(
        _comb1_body, grid=(n // _BLK,),
        in_specs=[_acc_spec(), _row_spec(), _deg_spec(), _b_spec(), _w_spec()],
        out_specs=_row_spec(),
        out_shape=jax.ShapeDtypeStruct((n, _D), _f32),
    )(acc1, g1, deg1, b1.reshape(1, _D), W2)

    acc2 = _scat_e(rowf, rowf, colf, g2)

    gp = pl.pallas_call(
        _comb2_body, grid=(n // _BLK,),
        in_specs=[_acc_spec(), _row_spec(), _deg_spec(), _b_spec(), _w_spec(),
                  _b_spec(), _deg_spec()],
        out_specs=_row_spec(),
        out_shape=jax.ShapeDtypeStruct((n, _D), _f32),
    )(acc2, g2, deg1, b2.reshape(1, _D), Wp, bp.reshape(1, _D), deg2)

    acc3 = _scat_e2(rowf, colf, col2f, gp)

    g4 = pl.pallas_call(
        _h1_body, grid=(n // _BLK,),
        in_specs=[_acc_spec(), _deg_spec()],
        out_specs=_row_spec(),
        out_shape=jax.ShapeDtypeStruct((n, _D), _f32),
    )(acc3, deg2)

    acc4 = _scat_e2(rowf, colf, col2f, g4)

    zs = pl.pallas_call(
        _h2_body, grid=(n // _BLK,),
        in_specs=[_acc_spec()],
        out_specs=_row_spec(),
        out_shape=jax.ShapeDtypeStruct((n, _D), _f32),
    )(acc4)

    bidx = batch.astype(_i32)
    idx = jnp.concatenate(
        [bidx[:, 0], bidx[:, 1], bidx[:, 2],
         jnp.zeros((_BPAD - 3 * n,), _i32)]).reshape(_NW, _BPAD // _NW // _CH, _CH)
    gath = _kernel_g(zs, idx)
    return _loss(gath, n)


# speedup vs baseline: 1.0123x; 1.0123x over previous
import functools

import jax
import jax.numpy as jnp
from jax import lax
from jax.experimental import pallas as pl
from jax.experimental.pallas import tpu as pltpu
from jax.experimental.pallas import tpu_sc as plsc

_N = 10000
_D = 128
_E = 320000
_E2 = 640000
_NPAD = 10240
_TRASH = _N
_CH = 80
_NC = 2
_NS = 16
_NW = _NC * _NS
_DSEG = _NPAD // _NS
_BPAD = 30720

_f32 = jnp.float32
_i32 = jnp.int32

_MESH = plsc.VectorSubcoreMesh(core_axis_name="c", subcore_axis_name="s")


def _zero_vec(ref, nelem):
    def zb(i, carry):
        ref[pl.ds(i * 16, 16)] = jnp.zeros((16,), _f32)
        return carry
    lax.fori_loop(0, nelem // 16, zb, None)


def _body_a(cd, degp, cs, ones_v, zdeg, deg_sh, ds0, ds1):
    c = lax.axis_index("c")
    s = lax.axis_index("s")
    wid = c * _NS + s
    rpt = _E // _NW // _CH
    pltpu.sync_copy(cd.at[wid], cs)
    for j in range(_CH // 16):
        ones_v[pl.ds(j * 16, 16)] = jnp.ones((16,), _f32)
    _zero_vec(zdeg, _DSEG)
    pltpu.sync_copy(zdeg, deg_sh.at[pl.ds(s * _DSEG, _DSEG)])
    plsc.subcore_barrier()

    pltpu.async_copy(ones_v, deg_sh.at[cs.at[0]], ds0, add=True)
    pltpu.async_copy(ones_v, deg_sh.at[cs.at[1]], ds1, add=True)

    def pair(k, carry):
        i2 = 2 * k + 2
        i3 = 2 * k + 3

        @pl.when(i2 < rpt)
        def _():
            pltpu.make_async_copy(ones_v, deg_sh.at[cs.at[0]], ds0).wait()
            pltpu.async_copy(ones_v, deg_sh.at[cs.at[i2]], ds0, add=True)

        @pl.when(i3 < rpt)
        def _():
            pltpu.make_async_copy(ones_v, deg_sh.at[cs.at[0]], ds1).wait()
            pltpu.async_copy(ones_v, deg_sh.at[cs.at[i3]], ds1, add=True)

        return carry

    lax.fori_loop(0, rpt // 2, pair, None)
    pltpu.make_async_copy(ones_v, deg_sh.at[cs.at[0]], ds0).wait()
    pltpu.make_async_copy(ones_v, deg_sh.at[cs.at[0]], ds1).wait()
    plsc.subcore_barrier()
    pltpu.sync_copy(deg_sh.at[pl.ds(s * _DSEG, _DSEG)],
                    degp.at[pl.ds(c * _NPAD + s * _DSEG, _DSEG)])


@functools.partial(
    pl.kernel,
    out_type=jax.ShapeDtypeStruct((_NC * _NPAD,), _f32),
    mesh=_MESH,
    scratch_types=[
        pltpu.VMEM((_E // _NW // _CH, _CH), _i32),
        pltpu.VMEM((_CH,), _f32),
        pltpu.VMEM((_DSEG,), _f32),
        pltpu.VMEM_SHARED((_NPAD,), _f32),
        pltpu.SemaphoreType.DMA,
        pltpu.SemaphoreType.DMA,
    ],
)
def _kernel_a(cd, degp, *rest):
    _body_a(cd, degp, *rest)


def _body_b(rd, cd, mwin, col2, deg2p, rs, cs, c2s, c2r, key0, win0, key1,
            win1, keep0, keep1, zdeg, deg_sh, gs0, gs1, ds0, ds1, ds2, ds3):
    c = lax.axis_index("c")
    s = lax.axis_index("s")
    wid = c * _NS + s
    rpt = _E // _NW // _CH
    pltpu.sync_copy(rd.at[wid], rs)
    pltpu.sync_copy(cd.at[wid], cs)
    _zero_vec(zdeg, _DSEG)
    pltpu.sync_copy(zdeg, deg_sh.at[pl.ds(s * _DSEG, _DSEG)])
    plsc.subcore_barrier()
    ebase = wid * (_E // _NW)

    def do_gather(i, key_v, win_v, gsem):
        for j in range(_CH // 16):
            r16 = rs[i, pl.ds(j * 16, 16)]
            c16 = cs[i, pl.ds(j * 16, 16)]
            key_v[pl.ds(j * 16, 16)] = (jnp.minimum(r16, c16) * _N
                                        + jnp.maximum(r16, c16))
        pltpu.async_copy(mwin.at[key_v], win_v, gsem)

    def process(i, win_v, keep_v, dsc, dsr):
        for j in range(_CH // 16):
            r16 = rs[i, pl.ds(j * 16, 16)]
            c16 = cs[i, pl.ds(j * 16, 16)]
            w16 = win_v[pl.ds(j * 16, 16)]
            e16 = lax.iota(_i32, 16) + (ebase + i * _CH + j * 16)
            keep16 = (w16 == e16) & (r16 != c16)
            c2s[pl.ds(i * _CH + j * 16, 16)] = jnp.where(keep16, c16, _TRASH)
            c2r[pl.ds(i * _CH + j * 16, 16)] = jnp.where(keep16, r16, _TRASH)
            keep_v[pl.ds(j * 16, 16)] = jnp.where(keep16, 1.0, 0.0).astype(_f32)
        pltpu.async_copy(keep_v, deg_sh.at[cs.at[i]], dsc, add=True)
        pltpu.async_copy(keep_v, deg_sh.at[rs.at[i]], dsr, add=True)

    do_gather(0, key0, win0, gs0)
    do_gather(1, key1, win1, gs1)

    def pair(k, carry):
        i0 = 2 * k
        i1 = 2 * k + 1
        i2 = 2 * k + 2
        i3 = 2 * k + 3

        @pl.when(k > 0)
        def _():
            pltpu.make_async_copy(keep0, deg_sh.at[cs.at[0]], ds0).wait()
            pltpu.make_async_copy(keep0, deg_sh.at[cs.at[0]], ds2).wait()
            pltpu.make_async_copy(keep1, deg_sh.at[cs.at[0]], ds1).wait()
            pltpu.make_async_copy(keep1, deg_sh.at[cs.at[0]], ds3).wait()

        pltpu.make_async_copy(mwin.at[key0], win0, gs0).wait()
        process(i0, win0, keep0, ds0, ds2)

        @pl.when(i2 < rpt)
        def _():
            do_gather(i2, key0, win0, gs0)

        pltpu.make_async_copy(mwin.at[key1], win1, gs1).wait()
        process(i1, win1, keep1, ds1, ds3)

        @pl.when(i3 < rpt)
        def _():
            do_gather(i3, key1, win1, gs1)

        return carry

    lax.fori_loop(0, rpt // 2, pair, None)
    pltpu.make_async_copy(keep0, deg_sh.at[cs.at[0]], ds0).wait()
    pltpu.make_async_copy(keep0, deg_sh.at[cs.at[0]], ds2).wait()
    pltpu.make_async_copy(mwin.at[key0], win0, gs0).wait()
    process(rpt - 1, win0, keep0, ds0, ds2)
    pltpu.make_async_copy(keep0, deg_sh.at[cs.at[0]], ds0).wait()
    pltpu.make_async_copy(keep0, deg_sh.at[cs.at[0]], ds2).wait()
    pltpu.make_async_copy(keep1, deg_sh.at[cs.at[0]], ds1).wait()
    pltpu.make_async_copy(keep1, deg_sh.at[cs.at[0]], ds3).wait()
    ept = _E // _NW
    pltpu.sync_copy(c2s, col2.at[pl.ds(wid * ept, ept)])
    pltpu.sync_copy(c2r, col2.at[pl.ds(_E + wid * ept, ept)])
    plsc.subcore_barrier()
    pltpu.sync_copy(deg_sh.at[pl.ds(s * _DSEG, _DSEG)],
                    deg2p.at[pl.ds(c * _NPAD + s * _DSEG, _DSEG)])


@functools.partial(
    pl.kernel,
    out_type=(
        jax.ShapeDtypeStruct((_E2,), _i32),
        jax.ShapeDtypeStruct((_NC * _NPAD,), _f32),
    ),
    mesh=_MESH,
    scratch_types=[
        pltpu.VMEM((_E // _NW // _CH, _CH), _i32),
        pltpu.VMEM((_E // _NW // _CH, _CH), _i32),
        pltpu.VMEM((_E // _NW,), _i32),
        pltpu.VMEM((_E // _NW,), _i32),
        pltpu.VMEM((_CH,), _i32),
        pltpu.VMEM((_CH,), _i32),
        pltpu.VMEM((_CH,), _i32),
        pltpu.VMEM((_CH,), _i32),
        pltpu.VMEM((_CH,), _f32),
        pltpu.VMEM((_CH,), _f32),
        pltpu.VMEM((_DSEG,), _f32),
        pltpu.VMEM_SHARED((_NPAD,), _f32),
        pltpu.SemaphoreType.DMA,
        pltpu.SemaphoreType.DMA,
        pltpu.SemaphoreType.DMA,
        pltpu.SemaphoreType.DMA,
        pltpu.SemaphoreType.DMA,
        pltpu.SemaphoreType.DMA,
    ],
)
def _kernel_b(rd, cd, mwin, col2, deg2p, *rest):
    _body_b(rd, cd, mwin, col2, deg2p, *rest)


def _make_scat(n_edges, split_rows, emit_win=False):
    ept = n_edges // _NW
    rpt = ept // _CH

    def body(rlo, rhi, colf, g, accp, *rest):
        if emit_win:
            (mwin, rv0, cv0, rv1, cv1, buf0, buf1, zrows, acc_sh, gs0, gs1,
             ss0, ss1, key0, ids0, key1, ids1, ws0, ws1) = rest
        else:
            (rv0, cv0, rv1, cv1, buf0, buf1, zrows, acc_sh,
             gs0, gs1, ss0, ss1) = rest
        c = lax.axis_index("c")
        s = lax.axis_index("s")
        wid = c * _NS + s

        def zr(i, carry):
            for j in range(_D // 16):
                zrows[i, pl.ds(j * 16, 16)] = jnp.zeros((16,), _f32)
            return carry

        lax.fori_loop(0, 16, zr, None)
        for k in range(_DSEG // 16):
            pltpu.sync_copy(zrows, acc_sh.at[pl.ds(s * _DSEG + k * 16, 16)])
        plsc.subcore_barrier()
        ebase = wid * ept

        def load_and_gather(i, rv, cv, buf, gsem):
            if split_rows:
                hbase = s * ept + i * _CH

                @pl.when(c == 0)
                def _():
                    pltpu.async_copy(rlo.at[pl.ds(hbase, _CH)], rv, gsem)

                @pl.when(c == 1)
                def _():
                    pltpu.async_copy(rhi.at[pl.ds(hbase, _CH)], rv, gsem)
            else:
                pltpu.async_copy(rlo.at[pl.ds(ebase + i * _CH, _CH)], rv, gsem)
            pltpu.async_copy(colf.at[pl.ds(ebase + i * _CH, _CH)], cv, gsem)
            pltpu.make_async_copy(rlo.at[pl.ds(ebase, _CH)], rv, gsem).wait()
            pltpu.make_async_copy(colf.at[pl.ds(ebase, _CH)], cv, gsem).wait()
            pltpu.async_copy(g.at[rv], buf, gsem)

        def win_scatter(i, rv, cv, key_v, ids_v, wsem):
            for j in range(_CH // 16):
                r16 = rv[pl.ds(j * 16, 16)]
                c16 = cv[pl.ds(j * 16, 16)]
                key_v[pl.ds(j * 16, 16)] = (jnp.minimum(r16, c16) * _N
                                            + jnp.maximum(r16, c16))
                ids_v[pl.ds(j * 16, 16)] = (lax.iota(_i32, 16)
                                            + (ebase + i * _CH + j * 16))
            pltpu.async_copy(ids_v, mwin.at[key_v], wsem)

        load_and_gather(0, rv0, cv0, buf0, gs0)
        load_and_gather(1, rv1, cv1, buf1, gs1)

        def pair(k, carry):
            i0 = 2 * k
            i1 = 2 * k + 1
            i2 = 2 * k + 2
            i3 = 2 * k + 3
            if emit_win:
                @pl.when(k > 0)
                def _():
                    pltpu.make_async_copy(ids0, mwin.at[key0], ws0).wait()
                    pltpu.make_async_copy(ids1, mwin.at[key1], ws1).wait()

            pltpu.make_async_copy(g.at[rv0], buf0, gs0).wait()
            pltpu.async_copy(buf0, acc_sh.at[cv0], ss0, add=True)
            if emit_win:
                win_scatter(i0, rv0, cv0, key0, ids0, ws0)
            pltpu.make_async_copy(g.at[rv1], buf1, gs1).wait()
            pltpu.async_copy(buf1, acc_sh.at[cv1], ss1, add=True)
            if emit_win:
                win_scatter(i1, rv1, cv1, key1, ids1, ws1)

            @pl.when(i2 < rpt)
            def _():
                pltpu.make_async_copy(buf0, acc_sh.at[cv0], ss0).wait()
                load_and_gather(i2, rv0, cv0, buf0, gs0)

            @pl.when(i3 < rpt)
            def _():
                pltpu.make_async_copy(buf1, acc_sh.at[cv1], ss1).wait()
                load_and_gather(i3, rv1, cv1, buf1, gs1)

            return carry

        lax.fori_loop(0, rpt // 2, pair, None)
        if rpt % 2 == 1:
            pltpu.make_async_copy(g.at[rv0], buf0, gs0).wait()
            pltpu.sync_copy(buf0, acc_sh.at[cv0], add=True)
            pltpu.make_async_copy(buf1, acc_sh.at[cv1], ss1).wait()
            if emit_win:
                pltpu.make_async_copy(ids0, mwin.at[key0], ws0).wait()
                win_scatter(rpt - 1, rv0, cv0, key0, ids0, ws0)
        else:
            pltpu.make_async_copy(buf0, acc_sh.at[cv0], ss0).wait()
            pltpu.make_async_copy(buf1, acc_sh.at[cv1], ss1).wait()
        if emit_win:
            pltpu.make_async_copy(ids0, mwin.at[key0], ws0).wait()
            pltpu.make_async_copy(ids1, mwin.at[key1], ws1).wait()

        plsc.subcore_barrier()
        pltpu.sync_copy(acc_sh.at[pl.ds(s * _DSEG, _DSEG)],
                        accp.at[c, pl.ds(s * _DSEG, _DSEG)])

    out_type = jax.ShapeDtypeStruct((_NC, _NPAD, _D), _f32)
    scratch = [
        pltpu.VMEM((_CH,), _i32),
        pltpu.VMEM((_CH,), _i32),
        pltpu.VMEM((_CH,), _i32),
        pltpu.VMEM((_CH,), _i32),
        pltpu.VMEM((_CH, _D), _f32),
        pltpu.VMEM((_CH, _D), _f32),
        pltpu.VMEM((16, _D), _f32),
        pltpu.VMEM_SHARED((_NPAD, _D), _f32),
        pltpu.SemaphoreType.DMA,
        pltpu.SemaphoreType.DMA,
        pltpu.SemaphoreType.DMA,
        pltpu.SemaphoreType.DMA,
    ]
    if emit_win:
        out_type = (out_type, jax.ShapeDtypeStruct((_N * _N,), _i32))
        scratch = scratch + [
            pltpu.VMEM((_CH,), _i32),
            pltpu.VMEM((_CH,), _i32),
            pltpu.VMEM((_CH,), _i32),
            pltpu.VMEM((_CH,), _i32),
            pltpu.SemaphoreType.DMA,
            pltpu.SemaphoreType.DMA,
        ]
    return pl.kernel(
        body,
        out_type=out_type,
        mesh=_MESH,
        scratch_types=scratch,
    )


_scat_e_win = _make_scat(_E, False, emit_win=True)
_scat_e = _make_scat(_E, False)
_scat_e2 = _make_scat(_E2, True)


def _body_g(src, idxd, out, is_, buf, sem):
    c = lax.axis_index("c")
    s = lax.axis_index("s")
    wid = c * _NS + s
    rpt = _BPAD // _NW // _CH
    slab0 = wid * rpt
    pltpu.sync_copy(idxd.at[wid], is_)

    def body(i, carry):
        pltpu.async_copy(src.at[is_.at[i]], buf, sem).wait()
        pltpu.sync_copy(buf, out.at[pl.ds((slab0 + i) * _CH, _CH)])
        return carry

    lax.fori_loop(0, rpt, body, None)


@functools.partial(
    pl.kernel,
    out_type=jax.ShapeDtypeStruct((_BPAD, _D), _f32),
    mesh=_MESH,
    scratch_types=[
        pltpu.VMEM((_BPAD // _NW // _CH, _CH), _i32),
        pltpu.VMEM((_CH, _D), _f32),
        pltpu.SemaphoreType.DMA,
    ],
)
def _kernel_g(src, idxd, out, *rest):
    _body_g(src, idxd, out, *rest)


_BLK = 2000


def _mm1_body(x_ref, w_ref, deg_ref, o_ref):
    dinv = lax.rsqrt(deg_ref[...])
    o_ref[...] = dinv * jnp.dot(
        x_ref[...], w_ref[...], preferred_element_type=_f32)


def _comb1_body(acc_ref, g1_ref, deg_ref, b_ref, w_ref, o_ref):
    dinv = lax.rsqrt(deg_ref[...])
    sacc = acc_ref[0] + acc_ref[1] + g1_ref[...]
    u = jnp.maximum(dinv * sacc + b_ref[...], 0.0)
    o_ref[...] = dinv * jnp.dot(
        u, w_ref[...], preferred_element_type=_f32)


def _comb2_body(acc_ref, g2_ref, deg_ref, b2_ref, wp_ref, bp_ref, deg2_ref, o_ref):
    dinv = lax.rsqrt(deg_ref[...])
    deg2 = deg2_ref[...]
    dinv2 = jnp.where(deg2 > 0, lax.rsqrt(deg2), 0.0)
    sacc = acc_ref[0] + acc_ref[1] + g2_ref[...]
    v = dinv * sacc + b2_ref[...]
    nodeb = jnp.dot(v, wp_ref[...], preferred_element_type=_f32) + bp_ref[...]
    o_ref[...] = dinv2 * nodeb


def _h1_body(acc_ref, deg2_ref, o_ref):
    deg2 = deg2_ref[...]
    ideg2 = jnp.where(deg2 > 0, 1.0 / deg2, 0.0)
    o_ref[...] = ideg2 * (acc_ref[0] + acc_ref[1])


def _h2_body(acc_ref, o_ref):
    o_ref[...] = acc_ref[0] + acc_ref[1]


def _row_spec():
    return pl.BlockSpec((_BLK, _D), lambda i: (i, 0))


def _acc_spec():
    return pl.BlockSpec((_NC, _BLK, _D), lambda i: (0, i, 0))


def _w_spec():
    return pl.BlockSpec((_D, _D), lambda i: (0, 0))


def _b_spec():
    return pl.BlockSpec((1, _D), lambda i: (0, 0))


def _deg_spec():
    return pl.BlockSpec((_BLK, 1), lambda i: (i, 0))


def _loss_body(a_ref, p_ref, g_ref, out_ref):
    i = pl.program_id(0)
    a = a_ref[...]
    p = p_ref[...]
    g = g_ref[...]
    na = jnp.maximum(jnp.sqrt(jnp.sum(a * a, axis=-1, keepdims=True)), 1e-8)
    npp = jnp.maximum(jnp.sqrt(jnp.sum(p * p, axis=-1, keepdims=True)), 1e-8)
    ng = jnp.maximum(jnp.sqrt(jnp.sum(g * g, axis=-1, keepdims=True)), 1e-8)
    cx = jnp.sum(a * p, axis=-1, keepdims=True) / (na * npp)
    cy = jnp.sum(a * g, axis=-1, keepdims=True) / (na * ng)
    li = jnp.log(1.0 + jnp.exp((cy - cx) / 0.2))

    @pl.when(i == 0)
    def _():
        out_ref[0, 0] = 0.0

    out_ref[0, 0] += jnp.sum(li)


def _loss(gath, b):
    nb = _N // _BLK
    out = pl.pallas_call(
        _loss_body,
        grid=(nb,),
        in_specs=[
            pl.BlockSpec((_BLK, _D), lambda i: (i, 0)),
            pl.BlockSpec((_BLK, _D), lambda i: (i + nb, 0)),
            pl.BlockSpec((_BLK, _D), lambda i: (i + 2 * nb, 0)),
        ],
        out_specs=pl.BlockSpec(memory_space=pltpu.SMEM),
        out_shape=jax.ShapeDtypeStruct((1, 1), _f32),
    )(gath, gath, gath)
    return out[0, 0] / b


def kernel(x, edge_index, batch, W1, b1, W2, b2, Wp, bp):
    n = _N
    ei = edge_index.astype(_i32)
    rd = ei[0].reshape(_NW, _E // _NW // _CH, _CH)
    cd = ei[1].reshape(_NW, _E // _NW // _CH, _CH)
    rowf = ei[0]
    colf = ei[1]

    degp = _kernel_a(cd)
    degp = degp.reshape(_NC, _NPAD)
    deg1 = (degp[0, :n] + degp[1, :n] + 1.0).reshape(n, 1)

    g1 = pl.pallas_call(
        _mm1_body, grid=(n // _BLK,),
        in_specs=[_row_spec(), _w_spec(), _deg_spec()],
        out_specs=_row_spec(),
        out_shape=jax.ShapeDtypeStruct((n, _D), _f32),
    )(x, W1, deg1)

    acc1, mwin = _scat_e_win(rowf, rowf, colf, g1)

    col2f, deg2p = _kernel_b(rd, cd, mwin)
    deg2p = deg2p.reshape(_NC, _NPAD)
    deg2 = (deg2p[0, :n] + deg2p[1, :n]).reshape(n, 1)

    g2 = pl.pallas_call(
        _comb1_body, grid=(n // _BLK,),
        in_specs=[_acc_spec(), _row_spec(), _deg_spec(), _b_spec(), _w_spec()],
        out_specs=_row_spec(),
        out_shape=jax.ShapeDtypeStruct((n, _D), _f32),
    )(acc1, g1, deg1, b1.reshape(1, _D), W2)

    acc2 = _scat_e(rowf, rowf, colf, g2)

    gp = pl.pallas_call(
        _comb2_body, grid=(n // _BLK,),
        in_specs=[_acc_spec(), _row_spec(), _deg_spec(), _b_spec(), _w_spec(),
                  _b_spec(), _deg_spec()],
        out_specs=_row_spec(),
        out_shape=jax.ShapeDtypeStruct((n, _D), _f32),
    )(acc2, g2, deg1, b2.reshape(1, _D), Wp, bp.reshape(1, _D), deg2)

    acc3 = _scat_e2(rowf, colf, col2f, gp)

    g4 = pl.pallas_call(
        _h1_body, grid=(n // _BLK,),
        in_specs=[_acc_spec(), _deg_spec()],
        out_specs=_row_spec(),
        out_shape=jax.ShapeDtypeStruct((n, _D), _f32),
    )(acc3, deg2)

    acc4 = _scat_e2(rowf, colf, col2f, g4)

    zs = pl.pallas_call(
        _h2_body, grid=(n // _BLK,),
        in_specs=[_acc_spec()],
        out_specs=_row_spec(),
        out_shape=jax.ShapeDtypeStruct((n, _D), _f32),
    )(acc4)

    bidx = batch.astype(_i32)
    idx = jnp.concatenate(
        [bidx[:, 0], bidx[:, 1], bidx[:, 2],
         jnp.zeros((_BPAD - 3 * n,), _i32)]).reshape(_NW, _BPAD // _NW // _CH, _CH)
    gath = _kernel_g(zs, idx)
    return _loss(gath, n)
